# baseline (speedup 1.0000x reference)
import functools

import jax
import jax.numpy as jnp
from jax import lax
from jax.experimental import pallas as pl
from jax.experimental.pallas import tpu as pltpu
from jax.experimental.pallas import tpu_sc as plsc

N = 10000
E = 320000
D = 128

NC = 2
NS = 16
NW = NC * NS
EW = E // NW

B = 50
K = EW // B
G = 8
NG = K // G
NSLOT = 3
NBUF = 6

HB_B = 125
HB_K = EW // HB_B
HB_BATCH = 16

RPT = 624
REM = N - NS * RPT

NPAD = 10240
HB = NPAD // NS

_mesh = plsc.VectorSubcoreMesh(core_axis_name="c", subcore_axis_name="s")


@functools.partial(
    pl.kernel,
    out_type=jax.ShapeDtypeStruct((NC, NPAD), jnp.float32),
    mesh=_mesh,
    scratch_types=[
        pltpu.VMEM((HB_K, HB_B), jnp.int32),
        pltpu.VMEM((HB_B,), jnp.float32),
        pltpu.VMEM_SHARED((NPAD,), jnp.float32),
        pltpu.SemaphoreType.DMA,
    ],
)
def _degree_kernel(dst_hbm, ones_hbm, zeros_hbm, out_hbm,
                   idx_v, ones_v, deg_sh, hsem):
    cid = lax.axis_index("c")
    sid = lax.axis_index("s")
    wid = cid * NS + sid
    pltpu.sync_copy(dst_hbm.at[pl.ds(wid * HB_K, HB_K)], idx_v)
    pltpu.sync_copy(ones_hbm, ones_v)
    pltpu.sync_copy(zeros_hbm.at[pl.ds(sid * HB, HB)],
                    deg_sh.at[pl.ds(sid * HB, HB)])
    plsc.subcore_barrier()

    for base in range(0, HB_K, HB_BATCH):
        for j in range(base, base + HB_BATCH):
            pltpu.async_copy(ones_v, deg_sh.at[idx_v.at[j]], hsem, add=True)
        for j in range(base, base + HB_BATCH):
            pltpu.make_async_copy(ones_v, deg_sh.at[idx_v.at[base]],
                                  hsem).wait()

    plsc.subcore_barrier()
    pltpu.sync_copy(deg_sh.at[pl.ds(sid * HB, HB)],
                    out_hbm.at[cid, pl.ds(sid * HB, HB)])


@functools.partial(
    pl.kernel,
    out_type=jax.ShapeDtypeStruct((NC, N, D), jnp.float32),
    mesh=_mesh,
    scratch_types=[
        pltpu.VMEM((NSLOT, G, B), jnp.int32),
        pltpu.VMEM((NSLOT, G, B), jnp.int32),
        pltpu.VMEM((NBUF, B, D), jnp.float32),
        pltpu.VMEM_SHARED((N, D), jnp.float32),
        pltpu.SemaphoreType.DMA,
        pltpu.SemaphoreType.DMA,
        pltpu.SemaphoreType.DMA,
    ],
)
def _propagate_kernel(g_hbm, src_hbm, dst_hbm, zeros_hbm, out_hbm,
                      sidx, didx, rows_v, acc_sh, gsem, ssem, isem):
    cid = lax.axis_index("c")
    sid = lax.axis_index("s")
    wid = cid * NS + sid

    pltpu.sync_copy(src_hbm.at[wid, pl.ds(0, G)], sidx.at[0])
    pltpu.sync_copy(dst_hbm.at[wid, pl.ds(0, G)], didx.at[0])

    @pl.when(cid == 0)
    def _seed_g():
        pltpu.sync_copy(g_hbm.at[pl.ds(sid * RPT, RPT)],
                        acc_sh.at[pl.ds(sid * RPT, RPT)])

        @pl.when(sid == NS - 1)
        def _seed_rem():
            pltpu.sync_copy(g_hbm.at[pl.ds(NS * RPT, REM)],
                            acc_sh.at[pl.ds(NS * RPT, REM)])

    @pl.when(cid == 1)
    def _seed_zero():
        pltpu.sync_copy(zeros_hbm.at[pl.ds(sid * RPT, RPT)],
                        acc_sh.at[pl.ds(sid * RPT, RPT)])

        @pl.when(sid == NS - 1)
        def _zero_rem():
            pltpu.sync_copy(zeros_hbm.at[pl.ds(NS * RPT, REM)],
                            acc_sh.at[pl.ds(NS * RPT, REM)])

    for b in range(NBUF - 1):
        pltpu.async_copy(g_hbm.at[sidx.at[0, b]], rows_v.at[b], gsem)
    pltpu.async_copy(src_hbm.at[wid, pl.ds(G, G)], sidx.at[1], isem)
    pltpu.async_copy(dst_hbm.at[wid, pl.ds(G, G)], didx.at[1], isem)

    plsc.subcore_barrier()

    def body(j, carry):
        m = lax.div(j, G)
        r = lax.rem(j, G)
        slot = lax.rem(m, NSLOT)
        buf = lax.rem(j, NBUF)

        pltpu.make_async_copy(g_hbm.at[sidx.at[slot, r]], rows_v.at[buf],
                              gsem).wait()

        @pl.when(j >= 1)
        def _drain_prev():
            pltpu.make_async_copy(rows_v.at[buf], acc_sh.at[didx.at[slot, r]],
                                  ssem).wait()

        pltpu.async_copy(rows_v.at[buf], acc_sh.at[didx.at[slot, r]], ssem,
                         add=True)

        @pl.when((r == 1) & (m + 2 < NG))
        def _prefetch_group():
            gm = m + 2
            gslot = lax.rem(gm, NSLOT)
            pltpu.async_copy(src_hbm.at[wid, pl.ds(gm * G, G)],
                             sidx.at[gslot], isem)
            pltpu.async_copy(dst_hbm.at[wid, pl.ds(gm * G, G)],
                             didx.at[gslot], isem)

        jn = j + NBUF - 1

        @pl.when(jn < K)
        def _refill():
            gm = lax.div(jn, G)
            gr = lax.rem(jn, G)
            gslot = lax.rem(gm, NSLOT)

            @pl.when((gr == 0) & (gm >= 1))
            def _wait_group():
                pltpu.make_async_copy(src_hbm.at[wid, pl.ds(0, G)],
                                      sidx.at[0], isem).wait()
                pltpu.make_async_copy(dst_hbm.at[wid, pl.ds(0, G)],
                                      didx.at[0], isem).wait()

            pltpu.async_copy(g_hbm.at[sidx.at[gslot, gr]],
                             rows_v.at[lax.rem(jn, NBUF)], gsem)

        return carry

    lax.fori_loop(0, K, body, 0)
    pltpu.make_async_copy(rows_v.at[0], acc_sh.at[didx.at[0, 0]], ssem).wait()

    plsc.subcore_barrier()
    pltpu.sync_copy(acc_sh.at[pl.ds(sid * RPT, RPT)],
                    out_hbm.at[cid, pl.ds(sid * RPT, RPT)])

    @pl.when(sid == NS - 1)
    def _write_rem():
        pltpu.sync_copy(acc_sh.at[pl.ds(NS * RPT, REM)],
                        out_hbm.at[cid, pl.ds(NS * RPT, REM)])


BN = 10000
GRID = N // BN


def _t1_body(degT_ref, x_ref, wpre_ref, bpre_ref, w1_ref, g1_ref):
    dinv = lax.rsqrt(degT_ref[:, 0] + degT_ref[:, 1] + 1.0)
    h0 = jnp.dot(x_ref[...], wpre_ref[...],
                 preferred_element_type=jnp.float32) + bpre_ref[...]
    y1 = jnp.dot(h0, w1_ref[...], preferred_element_type=jnp.float32)
    g1_ref[...] = y1 * dinv[:, None]


def _t2_body(degT_ref, part_ref, b1_ref, w2_ref, g2_ref):
    dinv = lax.rsqrt(degT_ref[:, 0] + degT_ref[:, 1] + 1.0)
    s = part_ref[0] + part_ref[1]
    h1 = jax.nn.relu(s * dinv[:, None] + b1_ref[...])
    y2 = jnp.dot(h1, w2_ref[...], preferred_element_type=jnp.float32)
    g2_ref[...] = y2 * dinv[:, None]


def _t3_body(degT_ref, part_ref, b2_ref, out_ref):
    dinv = lax.rsqrt(degT_ref[:, 0] + degT_ref[:, 1] + 1.0)
    s = part_ref[0] + part_ref[1]
    h2 = s * dinv[:, None] + b2_ref[...]
    nrm = jnp.sqrt(jnp.sum(h2 * h2, axis=-1, keepdims=True))
    out_ref[...] = h2 / jnp.maximum(nrm, 1e-12)


_deg_spec = pl.BlockSpec((BN, 2), lambda i: (i, 0))
_row_spec = pl.BlockSpec((BN, D), lambda i: (i, 0))
_w_spec = pl.BlockSpec((D, D), lambda i: (0, 0))
_b_spec = pl.BlockSpec((1, D), lambda i: (0, 0))
_part_spec = pl.BlockSpec((NC, BN, D), lambda i: (0, i, 0))

_t1 = pl.pallas_call(
    _t1_body,
    grid=(GRID,),
    in_specs=[_deg_spec, _row_spec, _w_spec, _b_spec, _w_spec],
    out_specs=_row_spec,
    out_shape=jax.ShapeDtypeStruct((N, D), jnp.float32),
)

_t2 = pl.pallas_call(
    _t2_body,
    grid=(GRID,),
    in_specs=[_deg_spec, _part_spec, _b_spec, _w_spec],
    out_specs=_row_spec,
    out_shape=jax.ShapeDtypeStruct((N, D), jnp.float32),
)

_t3 = pl.pallas_call(
    _t3_body,
    grid=(GRID,),
    in_specs=[_deg_spec, _part_spec, _b_spec],
    out_specs=_row_spec,
    out_shape=jax.ShapeDtypeStruct((N, D), jnp.float32),
)


def kernel(x, edge_index, W_pre, b_pre, W1, b1, W2, b2):
    src3d = edge_index[0].reshape(NW, K, B)
    dst3d = edge_index[1].reshape(NW, K, B)
    dst2d_h = edge_index[1].reshape(E // HB_B, HB_B)
    ones_b = jnp.ones((HB_B,), jnp.float32)
    zeros_np = jnp.zeros((NPAD,), jnp.float32)
    zeros_nd = jnp.zeros((N, D), jnp.float32)

    deg_part = _degree_kernel(dst2d_h, ones_b, zeros_np)
    degT = deg_part[:, :N].T

    g1 = _t1(degT, x, W_pre, b_pre.reshape(1, D), W1)
    p1 = _propagate_kernel(g1, src3d, dst3d, zeros_nd)
    g2 = _t2(degT, p1, b1.reshape(1, D), W2)
    p2 = _propagate_kernel(g2, src3d, dst3d, zeros_nd)
    return _t3(degT, p2, b2.reshape(1, D))


# speedup vs baseline: 1.0315x; 1.0315x over previous
import functools

import jax
import jax.numpy as jnp
from jax import lax
from jax.experimental import pallas as pl
from jax.experimental.pallas import tpu as pltpu
from jax.experimental.pallas import tpu_sc as plsc

N = 10000
E = 320000
D = 128

NC = 2
NS = 16
NW = NC * NS
EW = E // NW

B = 50
K = EW // B
G = 8
NG = K // G
NSLOT = 3
NBUF = 6

HB_B = 125
HB_K = EW // HB_B
HB_BATCH = 16

RPT = 624
REM = N - NS * RPT

NPAD = 10240
HB = NPAD // NS

_mesh = plsc.VectorSubcoreMesh(core_axis_name="c", subcore_axis_name="s")


@functools.partial(
    pl.kernel,
    out_type=jax.ShapeDtypeStruct((NC, NPAD), jnp.float32),
    mesh=_mesh,
    scratch_types=[
        pltpu.VMEM((HB_K, HB_B), jnp.int32),
        pltpu.VMEM((HB_B,), jnp.float32),
        pltpu.VMEM_SHARED((NPAD,), jnp.float32),
        pltpu.SemaphoreType.DMA,
    ],
)
def _degree_kernel(dst_hbm, ones_hbm, zeros_hbm, out_hbm,
                   idx_v, ones_v, deg_sh, hsem):
    cid = lax.axis_index("c")
    sid = lax.axis_index("s")
    wid = cid * NS + sid
    pltpu.sync_copy(dst_hbm.at[pl.ds(wid * HB_K, HB_K)], idx_v)
    pltpu.sync_copy(ones_hbm, ones_v)
    pltpu.sync_copy(zeros_hbm.at[pl.ds(sid * HB, HB)],
                    deg_sh.at[pl.ds(sid * HB, HB)])
    plsc.subcore_barrier()

    for base in range(0, HB_K, HB_BATCH):
        for j in range(base, base + HB_BATCH):
            pltpu.async_copy(ones_v, deg_sh.at[idx_v.at[j]], hsem, add=True)
        for j in range(base, base + HB_BATCH):
            pltpu.make_async_copy(ones_v, deg_sh.at[idx_v.at[base]],
                                  hsem).wait()

    plsc.subcore_barrier()
    pltpu.sync_copy(deg_sh.at[pl.ds(sid * HB, HB)],
                    out_hbm.at[cid, pl.ds(sid * HB, HB)])


@functools.partial(
    pl.kernel,
    out_type=jax.ShapeDtypeStruct((NC, N, D), jnp.float32),
    mesh=_mesh,
    scratch_types=[
        pltpu.VMEM((NSLOT, G, B), jnp.int32),
        pltpu.VMEM((NSLOT, G, B), jnp.int32),
        pltpu.VMEM((NBUF, B, D), jnp.float32),
        pltpu.VMEM_SHARED((N, D), jnp.float32),
        pltpu.SemaphoreType.DMA,
        pltpu.SemaphoreType.DMA,
        pltpu.SemaphoreType.DMA,
    ],
)
def _propagate_kernel(g_hbm, src_hbm, dst_hbm, zeros_hbm, out_hbm,
                      sidx, didx, rows_v, acc_sh, gsem, ssem, isem):
    cid = lax.axis_index("c")
    sid = lax.axis_index("s")
    wid = cid * NS + sid

    pltpu.sync_copy(src_hbm.at[wid, pl.ds(0, G)], sidx.at[0])
    pltpu.sync_copy(dst_hbm.at[wid, pl.ds(0, G)], didx.at[0])

    @pl.when(cid == 0)
    def _seed_g():
        pltpu.sync_copy(g_hbm.at[pl.ds(sid * RPT, RPT)],
                        acc_sh.at[pl.ds(sid * RPT, RPT)])

        @pl.when(sid == NS - 1)
        def _seed_rem():
            pltpu.sync_copy(g_hbm.at[pl.ds(NS * RPT, REM)],
                            acc_sh.at[pl.ds(NS * RPT, REM)])

    @pl.when(cid == 1)
    def _seed_zero():
        pltpu.sync_copy(zeros_hbm, acc_sh.at[pl.ds(sid * RPT, RPT)])

        @pl.when(sid == NS - 1)
        def _zero_rem():
            pltpu.sync_copy(zeros_hbm.at[pl.ds(0, REM)],
                            acc_sh.at[pl.ds(NS * RPT, REM)])

    for b in range(NBUF - 1):
        pltpu.async_copy(g_hbm.at[sidx.at[0, b]], rows_v.at[b], gsem)
    pltpu.async_copy(src_hbm.at[wid, pl.ds(G, G)], sidx.at[1], isem)
    pltpu.async_copy(dst_hbm.at[wid, pl.ds(G, G)], didx.at[1], isem)

    plsc.subcore_barrier()

    def body(j, carry):
        m = lax.div(j, G)
        r = lax.rem(j, G)
        slot = lax.rem(m, NSLOT)
        buf = lax.rem(j, NBUF)

        pltpu.make_async_copy(g_hbm.at[sidx.at[slot, r]], rows_v.at[buf],
                              gsem).wait()

        @pl.when(j >= 1)
        def _drain_prev():
            pltpu.make_async_copy(rows_v.at[buf], acc_sh.at[didx.at[slot, r]],
                                  ssem).wait()

        pltpu.async_copy(rows_v.at[buf], acc_sh.at[didx.at[slot, r]], ssem,
                         add=True)

        @pl.when((r == 1) & (m + 2 < NG))
        def _prefetch_group():
            gm = m + 2
            gslot = lax.rem(gm, NSLOT)
            pltpu.async_copy(src_hbm.at[wid, pl.ds(gm * G, G)],
                             sidx.at[gslot], isem)
            pltpu.async_copy(dst_hbm.at[wid, pl.ds(gm * G, G)],
                             didx.at[gslot], isem)

        jn = j + NBUF - 1

        @pl.when(jn < K)
        def _refill():
            gm = lax.div(jn, G)
            gr = lax.rem(jn, G)
            gslot = lax.rem(gm, NSLOT)

            @pl.when((gr == 0) & (gm >= 1))
            def _wait_group():
                pltpu.make_async_copy(src_hbm.at[wid, pl.ds(0, G)],
                                      sidx.at[0], isem).wait()
                pltpu.make_async_copy(dst_hbm.at[wid, pl.ds(0, G)],
                                      didx.at[0], isem).wait()

            pltpu.async_copy(g_hbm.at[sidx.at[gslot, gr]],
                             rows_v.at[lax.rem(jn, NBUF)], gsem)

        return carry

    lax.fori_loop(0, K, body, 0)
    pltpu.make_async_copy(rows_v.at[0], acc_sh.at[didx.at[0, 0]], ssem).wait()

    plsc.subcore_barrier()
    pltpu.sync_copy(acc_sh.at[pl.ds(sid * RPT, RPT)],
                    out_hbm.at[cid, pl.ds(sid * RPT, RPT)])

    @pl.when(sid == NS - 1)
    def _write_rem():
        pltpu.sync_copy(acc_sh.at[pl.ds(NS * RPT, REM)],
                        out_hbm.at[cid, pl.ds(NS * RPT, REM)])


BN = 5000
GRID = N // BN


def _t1_body(degT_ref, x_ref, wpre_ref, bpre_ref, w1_ref, g1_ref):
    dinv = lax.rsqrt(degT_ref[:, 0] + degT_ref[:, 1] + 1.0)
    h0 = jnp.dot(x_ref[...], wpre_ref[...],
                 preferred_element_type=jnp.float32) + bpre_ref[...]
    y1 = jnp.dot(h0, w1_ref[...], preferred_element_type=jnp.float32)
    g1_ref[...] = y1 * dinv[:, None]


def _t2_body(degT_ref, part_ref, b1_ref, w2_ref, g2_ref):
    dinv = lax.rsqrt(degT_ref[:, 0] + degT_ref[:, 1] + 1.0)
    s = part_ref[0] + part_ref[1]
    h1 = jax.nn.relu(s * dinv[:, None] + b1_ref[...])
    y2 = jnp.dot(h1, w2_ref[...], preferred_element_type=jnp.float32)
    g2_ref[...] = y2 * dinv[:, None]


def _t3_body(degT_ref, part_ref, b2_ref, out_ref):
    dinv = lax.rsqrt(degT_ref[:, 0] + degT_ref[:, 1] + 1.0)
    s = part_ref[0] + part_ref[1]
    h2 = s * dinv[:, None] + b2_ref[...]
    nrm = jnp.sqrt(jnp.sum(h2 * h2, axis=-1, keepdims=True))
    out_ref[...] = h2 / jnp.maximum(nrm, 1e-12)


_deg_spec = pl.BlockSpec((BN, 2), lambda i: (i, 0))
_row_spec = pl.BlockSpec((BN, D), lambda i: (i, 0))
_w_spec = pl.BlockSpec((D, D), lambda i: (0, 0))
_b_spec = pl.BlockSpec((1, D), lambda i: (0, 0))
_part_spec = pl.BlockSpec((NC, BN, D), lambda i: (0, i, 0))

_t1 = pl.pallas_call(
    _t1_body,
    grid=(GRID,),
    in_specs=[_deg_spec, _row_spec, _w_spec, _b_spec, _w_spec],
    out_specs=_row_spec,
    out_shape=jax.ShapeDtypeStruct((N, D), jnp.float32),
)

_t2 = pl.pallas_call(
    _t2_body,
    grid=(GRID,),
    in_specs=[_deg_spec, _part_spec, _b_spec, _w_spec],
    out_specs=_row_spec,
    out_shape=jax.ShapeDtypeStruct((N, D), jnp.float32),
)

_t3 = pl.pallas_call(
    _t3_body,
    grid=(GRID,),
    in_specs=[_deg_spec, _part_spec, _b_spec],
    out_specs=_row_spec,
    out_shape=jax.ShapeDtypeStruct((N, D), jnp.float32),
)


def kernel(x, edge_index, W_pre, b_pre, W1, b1, W2, b2):
    src3d = edge_index[0].reshape(NW, K, B)
    dst3d = edge_index[1].reshape(NW, K, B)
    dst2d_h = edge_index[1].reshape(E // HB_B, HB_B)
    ones_b = jnp.ones((HB_B,), jnp.float32)
    zeros_np = jnp.zeros((NPAD,), jnp.float32)
    zeros_st = jnp.zeros((RPT, D), jnp.float32)

    deg_part = _degree_kernel(dst2d_h, ones_b, zeros_np)
    degT = deg_part[:, :N].T

    g1 = _t1(degT, x, W_pre, b_pre.reshape(1, D), W1)
    p1 = _propagate_kernel(g1, src3d, dst3d, zeros_st)
    g2 = _t2(degT, p1, b1.reshape(1, D), W2)
    p2 = _propagate_kernel(g2, src3d, dst3d, zeros_st)
    return _t3(degT, p2, b2.reshape(1, D))
